# baseline (speedup 1.0000x reference)
import functools

import jax
import jax.numpy as jnp
from jax import lax
from jax.experimental import pallas as pl
from jax.experimental.pallas import tpu as pltpu
from jax.experimental.pallas import tpu_sc as plsc


def _sc_geometry():
    try:
        info = plsc.get_sparse_core_info()
        return info.num_cores, info.num_subcores
    except Exception:
        return 2, 16


@functools.lru_cache(maxsize=None)
def _make_gather_t(S, B, V, D, chunk):
    NC, NS = _sc_geometry()
    NW = NC * NS
    assert D == NW and B % chunk == 0 and chunk % 16 == 0
    n_chunks = B // chunk
    n_tiles = S * n_chunks
    assert n_tiles % 2 == 0
    pairs = n_tiles // 2
    mesh = plsc.VectorSubcoreMesh(core_axis_name="c", subcore_axis_name="s",
                                  num_cores=NC, num_subcores=NS)

    @functools.partial(
        pl.kernel,
        out_type=jax.ShapeDtypeStruct((S, D, B), jnp.float32),
        mesh=mesh,
        scratch_types=[
            pltpu.VMEM((V,), jnp.float32),
            pltpu.VMEM((chunk,), jnp.int32),
            pltpu.VMEM((chunk,), jnp.int32),
            pltpu.VMEM((chunk,), jnp.float32),
            pltpu.VMEM((chunk,), jnp.float32),
            pltpu.SemaphoreType.DMA,
            pltpu.SemaphoreType.DMA,
            pltpu.SemaphoreType.DMA,
            pltpu.SemaphoreType.DMA,
        ],
        compiler_params=pltpu.CompilerParams(use_tc_tiling_on_sc=True,
                                             needs_layout_passes=False,
                                             disable_bounds_checks=True),
    )
    def gather_kernel(uid_hbm, table_hbm, out_hbm, row_v, idx0, idx1,
                      res0, res1, si0, si1, so0, so1):
        d = lax.axis_index("s") * NC + lax.axis_index("c")
        row_copy = pltpu.make_async_copy(table_hbm.at[d, :], row_v, so0)
        row_copy.start()
        idx_v = (idx0, idx1)
        res_v = (res0, res1)
        si = (si0, si1)
        so = (so0, so1)

        def idx_copy(t, b):
            s = t // n_chunks
            b0 = (t % n_chunks) * chunk
            return pltpu.make_async_copy(
                uid_hbm.at[s, pl.ds(b0, chunk)], idx_v[b], si[b])

        def out_copy(t, b):
            s = t // n_chunks
            b0 = (t % n_chunks) * chunk
            return pltpu.make_async_copy(
                res_v[b], out_hbm.at[s, d, pl.ds(b0, chunk)], so[b])

        def compute(b):
            @plsc.parallel_loop(0, chunk, step=16, unroll=16)
            def _inner(off):
                iv = idx_v[b][pl.ds(off, 16)]
                res_v[b][pl.ds(off, 16)] = plsc.load_gather(row_v, [iv])

        idx_copy(0, 0).start()
        row_copy.wait()

        def body(p, _):
            t = 2 * p
            idx_copy(t, 0).wait()

            @pl.when(p > 0)
            def _():
                out_copy(t - 1, 1).wait()

            idx_copy(t + 1, 1).start()

            @pl.when(p > 0)
            def _():
                out_copy(t - 2, 0).wait()

            compute(0)
            out_copy(t, 0).start()
            idx_copy(t + 1, 1).wait()

            @pl.when(p < pairs - 1)
            def _():
                idx_copy(t + 2, 0).start()

            compute(1)
            out_copy(t + 1, 1).start()
            return _

        lax.fori_loop(0, pairs, body, None)
        out_copy(n_tiles - 2, 0).wait()
        out_copy(n_tiles - 1, 1).wait()

    return gather_kernel


def kernel(user_ids, table):
    Bv, S = user_ids.shape
    V, D = table.shape
    uid_t = user_ids.T.astype(jnp.int32)
    table_t = table.T
    out_t = _make_gather_t(S, Bv, V, D, 4096)(uid_t, table_t)
    return out_t.transpose(2, 0, 1)


# speedup vs baseline: 1.2315x; 1.2315x over previous
import functools

import jax
import jax.numpy as jnp
from jax import lax
from jax.experimental import pallas as pl
from jax.experimental.pallas import tpu as pltpu
from jax.experimental.pallas import tpu_sc as plsc


def _sc_geometry():
    try:
        info = plsc.get_sparse_core_info()
        return info.num_cores, info.num_subcores
    except Exception:
        return 2, 16


@functools.lru_cache(maxsize=None)
def _make_gather_t(S, B, V, D, cb):
    NC, NS = _sc_geometry()
    NW = NC * NS
    SG = 8
    full_groups = S // SG
    S_tail = S - full_groups * SG
    assert D == NW and B % cb == 0 and cb % 128 == 0 and cb % 16 == 0
    nb = B // cb
    n_blocks = full_groups * nb
    assert n_blocks % 2 == 0 and (S_tail * nb) % 2 == 0
    pairs = n_blocks // 2
    tail_pairs = (S_tail * nb) // 2
    mesh = plsc.VectorSubcoreMesh(core_axis_name="c", subcore_axis_name="s",
                                  num_cores=NC, num_subcores=NS)

    @functools.partial(
        pl.kernel,
        out_type=jax.ShapeDtypeStruct((S, D, B), jnp.float32),
        mesh=mesh,
        scratch_types=[
            pltpu.VMEM((V,), jnp.float32),
            pltpu.VMEM((SG, cb), jnp.int32),
            pltpu.VMEM((SG, cb), jnp.int32),
            pltpu.VMEM((cb,), jnp.int32),
            pltpu.VMEM((cb,), jnp.int32),
            pltpu.VMEM((cb,), jnp.float32),
            pltpu.VMEM((cb,), jnp.float32),
            pltpu.SemaphoreType.DMA,
            pltpu.SemaphoreType.DMA,
            pltpu.SemaphoreType.DMA,
            pltpu.SemaphoreType.DMA,
            pltpu.SemaphoreType.DMA,
        ],
        compiler_params=pltpu.CompilerParams(use_tc_tiling_on_sc=True,
                                             needs_layout_passes=False,
                                             disable_bounds_checks=True),
    )
    def gather_kernel(uid_hbm, table_hbm, out_hbm, row_v, blk0, blk1,
                      idx0, idx1, res0, res1, si0, si1, so0, so1, sr):
        d = lax.axis_index("s") * NC + lax.axis_index("c")
        row_copy = pltpu.make_async_copy(table_hbm.at[d, :], row_v, sr)
        row_copy.start()
        blk_v = (blk0, blk1)
        idx_v = (idx0, idx1)
        res_v = (res0, res1)
        si = (si0, si1)
        so = (so0, so1)

        def gather_into(load_iv, rb):
            @plsc.parallel_loop(0, cb, step=16, unroll=8)
            def _inner(off):
                iv = load_iv(off)
                res_v[rb][pl.ds(off, 16)] = plsc.load_gather(row_v, [iv])

        def out_copy(s, b0, rb):
            return pltpu.make_async_copy(
                res_v[rb], out_hbm.at[s, d, pl.ds(b0, cb)], so[rb])

        def blk_copy(t, b):
            g = t // nb
            b0 = (t % nb) * cb
            return pltpu.make_async_copy(
                uid_hbm.at[pl.ds(g * SG, SG), pl.ds(b0, cb)], blk_v[b], si[b])

        def process_block(t, b):
            g = t // nb
            b0 = (t % nb) * cb
            for r in range(SG):
                rb = r % 2
                s = g * SG + r
                if r >= 2:
                    out_copy(s, b0, rb).wait()
                gather_into(lambda off, _b=b, _r=r: blk_v[_b][_r, pl.ds(off, 16)], rb)
                out_copy(s, b0, rb).start()
            out_copy(g * SG, b0, 0).wait()
            out_copy(g * SG + 1, b0, 1).wait()

        blk_copy(0, 0).start()
        row_copy.wait()

        def body(p, _):
            t = 2 * p
            blk_copy(t, 0).wait()
            blk_copy(t + 1, 1).start()
            process_block(t, 0)
            blk_copy(t + 1, 1).wait()

            @pl.when(p < pairs - 1)
            def _():
                blk_copy(t + 2, 0).start()

            process_block(t + 1, 1)
            return _

        lax.fori_loop(0, pairs, body, None)

        s_base = full_groups * SG

        def idx_copy(t, b):
            s = s_base + t // nb
            b0 = (t % nb) * cb
            return pltpu.make_async_copy(
                uid_hbm.at[s, pl.ds(b0, cb)], idx_v[b], si[b])

        def out_copy2(t, rb):
            s = s_base + t // nb
            b0 = (t % nb) * cb
            return pltpu.make_async_copy(
                res_v[rb], out_hbm.at[s, d, pl.ds(b0, cb)], so[rb])

        idx_copy(0, 0).start()

        def body2(p, _):
            t = 2 * p
            idx_copy(t, 0).wait()

            @pl.when(p > 0)
            def _():
                out_copy2(t - 1, 1).wait()

            idx_copy(t + 1, 1).start()

            @pl.when(p > 0)
            def _():
                out_copy2(t - 2, 0).wait()

            gather_into(lambda off: idx_v[0][pl.ds(off, 16)], 0)
            out_copy2(t, 0).start()
            idx_copy(t + 1, 1).wait()

            @pl.when(p < tail_pairs - 1)
            def _():
                idx_copy(t + 2, 0).start()

            gather_into(lambda off: idx_v[1][pl.ds(off, 16)], 1)
            out_copy2(t + 1, 1).start()
            return _

        lax.fori_loop(0, tail_pairs, body2, None)
        out_copy2(S_tail * nb - 2, 0).wait()
        out_copy2(S_tail * nb - 1, 1).wait()

    return gather_kernel


def kernel(user_ids, table):
    Bv, S = user_ids.shape
    V, D = table.shape
    uid_t = user_ids.T.astype(jnp.int32)
    table_t = table.T
    out_t = _make_gather_t(S, Bv, V, D, 1024)(uid_t, table_t)
    return out_t.transpose(2, 0, 1)


# speedup vs baseline: 1.2329x; 1.0011x over previous
import functools

import jax
import jax.numpy as jnp
from jax import lax
from jax.experimental import pallas as pl
from jax.experimental.pallas import tpu as pltpu
from jax.experimental.pallas import tpu_sc as plsc


def _sc_geometry():
    try:
        info = plsc.get_sparse_core_info()
        return info.num_cores, info.num_subcores
    except Exception:
        return 2, 16


@functools.lru_cache(maxsize=None)
def _make_gather_t(S, B, V, D, cb):
    NC, NS = _sc_geometry()
    NW = NC * NS
    SG = 8
    full_groups = S // SG
    S_tail = S - full_groups * SG
    assert D == NW and B % cb == 0 and cb % 128 == 0 and cb % 16 == 0
    nb = B // cb
    n_blocks = full_groups * nb
    assert n_blocks % 2 == 0 and (S_tail * nb) % 2 == 0
    pairs = n_blocks // 2
    tail_pairs = (S_tail * nb) // 2
    mesh = plsc.VectorSubcoreMesh(core_axis_name="c", subcore_axis_name="s",
                                  num_cores=NC, num_subcores=NS)

    @functools.partial(
        pl.kernel,
        out_type=jax.ShapeDtypeStruct((S, D, B), jnp.float32),
        mesh=mesh,
        scratch_types=[
            pltpu.VMEM((V,), jnp.float32),
            pltpu.VMEM((SG, cb), jnp.int32),
            pltpu.VMEM((SG, cb), jnp.int32),
            pltpu.VMEM((cb,), jnp.int32),
            pltpu.VMEM((cb,), jnp.int32),
            pltpu.VMEM((cb,), jnp.float32),
            pltpu.VMEM((cb,), jnp.float32),
            pltpu.SemaphoreType.DMA,
            pltpu.SemaphoreType.DMA,
            pltpu.SemaphoreType.DMA,
            pltpu.SemaphoreType.DMA,
            pltpu.SemaphoreType.DMA,
        ],
        compiler_params=pltpu.CompilerParams(use_tc_tiling_on_sc=True,
                                             needs_layout_passes=False,
                                             disable_bounds_checks=True),
    )
    def gather_kernel(uid_hbm, table_hbm, out_hbm, row_v, blk0, blk1,
                      idx0, idx1, res0, res1, si0, si1, so0, so1, sr):
        d = lax.axis_index("s") * NC + lax.axis_index("c")
        row_copy = pltpu.make_async_copy(table_hbm.at[d, :], row_v, sr)
        row_copy.start()
        blk_v = (blk0, blk1)
        idx_v = (idx0, idx1)
        res_v = (res0, res1)
        si = (si0, si1)
        so = (so0, so1)

        def gather_into(load_iv, rb):
            @plsc.parallel_loop(0, cb, step=16, unroll=8)
            def _inner(off):
                iv = load_iv(off)
                res_v[rb][pl.ds(off, 16)] = plsc.load_gather(row_v, [iv])

        def out_copy(s, b0, rb):
            return pltpu.make_async_copy(
                res_v[rb], out_hbm.at[s, d, pl.ds(b0, cb)], so[rb])

        def blk_copy(t, b):
            g = t // nb
            b0 = (t % nb) * cb
            return pltpu.make_async_copy(
                uid_hbm.at[pl.ds(g * SG, SG), pl.ds(b0, cb)], blk_v[b], si[b])

        def process_block(t, b, guard):
            g = t // nb
            b0 = (t % nb) * cb
            for r in range(SG):
                rb = r % 2
                s = g * SG + r
                if r >= 2:
                    out_copy(s, b0, rb).wait()
                elif guard is None:
                    out_copy(s, b0, rb).wait()
                else:
                    @pl.when(guard)
                    def _(s=s, b0=b0, rb=rb):
                        out_copy(s, b0, rb).wait()
                gather_into(lambda off, _b=b, _r=r: blk_v[_b][_r, pl.ds(off, 16)], rb)
                out_copy(s, b0, rb).start()

        blk_copy(0, 0).start()
        row_copy.wait()

        def body(p, _):
            t = 2 * p
            blk_copy(t, 0).wait()
            blk_copy(t + 1, 1).start()
            process_block(t, 0, p > 0)
            blk_copy(t + 1, 1).wait()

            @pl.when(p < pairs - 1)
            def _():
                blk_copy(t + 2, 0).start()

            process_block(t + 1, 1, None)
            return _

        lax.fori_loop(0, pairs, body, None)
        out_copy(full_groups * SG - 2, B - cb, 0).wait()
        out_copy(full_groups * SG - 1, B - cb, 1).wait()

        s_base = full_groups * SG

        def idx_copy(t, b):
            s = s_base + t // nb
            b0 = (t % nb) * cb
            return pltpu.make_async_copy(
                uid_hbm.at[s, pl.ds(b0, cb)], idx_v[b], si[b])

        def out_copy2(t, rb):
            s = s_base + t // nb
            b0 = (t % nb) * cb
            return pltpu.make_async_copy(
                res_v[rb], out_hbm.at[s, d, pl.ds(b0, cb)], so[rb])

        idx_copy(0, 0).start()

        def body2(p, _):
            t = 2 * p
            idx_copy(t, 0).wait()

            @pl.when(p > 0)
            def _():
                out_copy2(t - 1, 1).wait()

            idx_copy(t + 1, 1).start()

            @pl.when(p > 0)
            def _():
                out_copy2(t - 2, 0).wait()

            gather_into(lambda off: idx_v[0][pl.ds(off, 16)], 0)
            out_copy2(t, 0).start()
            idx_copy(t + 1, 1).wait()

            @pl.when(p < tail_pairs - 1)
            def _():
                idx_copy(t + 2, 0).start()

            gather_into(lambda off: idx_v[1][pl.ds(off, 16)], 1)
            out_copy2(t + 1, 1).start()
            return _

        lax.fori_loop(0, tail_pairs, body2, None)
        out_copy2(S_tail * nb - 2, 0).wait()
        out_copy2(S_tail * nb - 1, 1).wait()

    return gather_kernel


def kernel(user_ids, table):
    Bv, S = user_ids.shape
    V, D = table.shape
    uid_t = user_ids.T.astype(jnp.int32)
    table_t = table.T
    out_t = _make_gather_t(S, Bv, V, D, 1024)(uid_t, table_t)
    return out_t.transpose(2, 0, 1)
